# Initial kernel scaffold; baseline (speedup 1.0000x reference)
import jax
import jax.numpy as jnp
from jax.experimental import pallas as pl


def kernel(inpt, cluster_centers):
    raise NotImplementedError("write your pallas kernel here")


# speedup vs baseline: 1.7191x; 1.7191x over previous
import jax
import jax.numpy as jnp
from jax.experimental import pallas as pl

_K = 8192
_KT = 2048
_P = 1024


def _labeller_kernel(x_ref, c_ref, out_ref):
    x = x_ref[0]
    x2 = jnp.sum(x * x, axis=1, keepdims=True)

    best = jnp.full((_P,), jnp.inf, dtype=jnp.float32)
    besti = jnp.zeros((_P,), dtype=jnp.int32)
    for t in range(_K // _KT):
        ct = c_ref[t * _KT:(t + 1) * _KT, :]
        c2 = jnp.sum(ct * ct, axis=1)
        dot = jax.lax.dot_general(
            x, ct, (((1,), (1,)), ((), ())),
            preferred_element_type=jnp.float32)
        sq = (x2 + c2[None, :]) - 2.0 * dot
        m = jnp.min(sq, axis=1)
        iota = jax.lax.broadcasted_iota(jnp.int32, (_P, _KT), 1)
        idx = jnp.min(
            jnp.where(sq == m[:, None], iota + t * _KT, _K),
            axis=1)
        upd = m < best
        best = jnp.where(upd, m, best)
        besti = jnp.where(upd, idx, besti)
    out_ref[0, 0, :] = besti


def kernel(inpt, cluster_centers):
    b, p, d = inpt.shape
    labels = pl.pallas_call(
        _labeller_kernel,
        grid=(b,),
        in_specs=[
            pl.BlockSpec((1, p, d), lambda i: (i, 0, 0)),
            pl.BlockSpec(cluster_centers.shape, lambda i: (0, 0)),
        ],
        out_specs=pl.BlockSpec((1, 1, p), lambda i: (i, 0, 0)),
        out_shape=jax.ShapeDtypeStruct((b, 1, p), jnp.int32),
    )(inpt, cluster_centers)
    return labels.reshape(b, p)


# speedup vs baseline: 2.0962x; 1.2193x over previous
import jax
import jax.numpy as jnp
from jax.experimental import pallas as pl

_K = 8192
_KT = 2048
_P = 1024
_L = 128


def _labeller_kernel(x_ref, c_ref, out_ref):
    x = x_ref[0]
    xm2 = -2.0 * x
    x2 = jnp.sum(x * x, axis=1, keepdims=True)

    lane = jax.lax.broadcasted_iota(jnp.int32, (_P, _L), 1)
    best = jnp.full((_P, _L), jnp.inf, dtype=jnp.float32)
    bestc = jnp.zeros((_P, _L), dtype=jnp.int32)
    for t in range(_K // _KT):
        ct = c_ref[t * _KT:(t + 1) * _KT, :]
        c2 = jnp.sum(ct * ct, axis=1)
        dm2 = jax.lax.dot_general(
            xm2, ct, (((1,), (1,)), ((), ())),
            preferred_element_type=jnp.float32)
        sq = (x2 + c2[None, :]) + dm2
        for c in range(_KT // _L):
            s = sq[:, c * _L:(c + 1) * _L]
            cid = t * (_KT // _L) + c
            upd = s < best
            best = jnp.where(upd, s, best)
            bestc = jnp.where(upd, jnp.full((_P, _L), cid, jnp.int32), bestc)

    idx = bestc * _L + lane
    m = jnp.min(best, axis=1, keepdims=True)
    out_ref[0, 0, :] = jnp.min(jnp.where(best == m, idx, _K), axis=1)


def kernel(inpt, cluster_centers):
    b, p, d = inpt.shape
    labels = pl.pallas_call(
        _labeller_kernel,
        grid=(b,),
        in_specs=[
            pl.BlockSpec((1, p, d), lambda i: (i, 0, 0)),
            pl.BlockSpec(cluster_centers.shape, lambda i: (0, 0)),
        ],
        out_specs=pl.BlockSpec((1, 1, p), lambda i: (i, 0, 0)),
        out_shape=jax.ShapeDtypeStruct((b, 1, p), jnp.int32),
    )(inpt, cluster_centers)
    return labels.reshape(b, p)
